# Initial kernel scaffold; baseline (speedup 1.0000x reference)
import jax
import jax.numpy as jnp
from jax.experimental import pallas as pl


def kernel(x, conv_w, conv_b, wih, bih, whh, bhh_n, fc_w, fc_b):
    raise NotImplementedError("write your pallas kernel here")


# speedup vs baseline: 1.3289x; 1.3289x over previous
import jax
import jax.numpy as jnp
from jax.experimental import pallas as pl
from jax.experimental.pallas import tpu as pltpu

C_IN = 12
C_OUT = 32
K = 5
PAD = 2
HIDDEN = 32
OUT = 5
OUT_PAD = 128
TB = 128


def _ecg_kernel(x_ref, cw_ref, cb_ref, wih_ref, bih_ref, whh_ref, bhhn_ref,
                fcw_ref, fcb_ref, out_ref, gx_scr):
    L, tb, _ = x_ref.shape
    T = L // 2
    H = HIDDEN
    G6 = 6 * H

    x = x_ref[...]
    zpad = jnp.zeros((PAD, tb, C_IN), jnp.float32)
    xp = jnp.concatenate([zpad, x, zpad], axis=0)
    taps = [xp[k:k + L] for k in range(K)]
    xi = jnp.concatenate(taps, axis=-1).reshape(L * tb, K * C_IN)
    y = jnp.dot(xi, cw_ref[...], preferred_element_type=jnp.float32) + cb_ref[...]
    y = jnp.maximum(y, 0.0)

    y4 = y.reshape(T, 2, tb, C_OUT)
    feats = jnp.maximum(y4[:, 0], y4[:, 1]).reshape(T * tb, C_OUT)

    gx = jnp.dot(feats, wih_ref[...],
                 preferred_element_type=jnp.float32) + bih_ref[...]
    gx_scr[...] = gx.reshape(T, tb, G6)

    whh = whh_ref[...]
    bhh_n = bhhn_ref[...]
    col = jax.lax.broadcasted_iota(jnp.int32, (1, G6), 1)
    is_bwd = (col // H) % 2 == 1

    def step(t, h):
        gf = gx_scr[t]
        gb = gx_scr[T - 1 - t]
        gx_t = jnp.where(is_bwd, gb, gf)
        gh = jnp.dot(h, whh, preferred_element_type=jnp.float32)
        rz = jax.nn.sigmoid(gx_t[:, :4 * H] + gh[:, :4 * H])
        r = rz[:, :2 * H]
        z = rz[:, 2 * H:]
        n = jnp.tanh(gx_t[:, 4 * H:] + r * (gh[:, 4 * H:] + bhh_n))
        return (1.0 - z) * n + z * h

    h0 = jnp.zeros((tb, 2 * H), jnp.float32)
    h = jax.lax.fori_loop(0, T, step, h0, unroll=4)

    out_ref[...] = jnp.dot(h, fcw_ref[...],
                           preferred_element_type=jnp.float32) + fcb_ref[...]


def kernel(x, conv_w, conv_b, wih, bih, whh, bhh_n, fc_w, fc_b):
    B, C, L = x.shape
    assert C == C_IN and L % 2 == 0
    tb = TB if B % TB == 0 else B
    G = B // tb
    xt = jnp.transpose(x, (2, 0, 1))

    args = (xt, conv_w, conv_b, wih, bih, whh, bhh_n, fc_w, fc_b)

    def full_spec(a):
        return pl.BlockSpec(a.shape, lambda g, nd=a.ndim: (0,) * nd)

    in_specs = ([pl.BlockSpec((L, tb, C_IN), lambda g: (0, g, 0))]
                + [full_spec(a) for a in args[1:]])

    out = pl.pallas_call(
        _ecg_kernel,
        out_shape=jax.ShapeDtypeStruct((B, OUT_PAD), jnp.float32),
        grid_spec=pltpu.PrefetchScalarGridSpec(
            num_scalar_prefetch=0,
            grid=(G,),
            in_specs=in_specs,
            out_specs=pl.BlockSpec((tb, OUT_PAD), lambda g: (g, 0)),
            scratch_shapes=[pltpu.VMEM((L // 2, tb, 6 * HIDDEN), jnp.float32)],
        ),
        compiler_params=pltpu.CompilerParams(
            dimension_semantics=("parallel",),
            vmem_limit_bytes=100 * 1024 * 1024,
        ),
    )(*args)
    return out[:, :OUT]


# speedup vs baseline: 5.4740x; 4.1192x over previous
import jax
import jax.numpy as jnp
from jax.experimental import pallas as pl
from jax.experimental.pallas import tpu as pltpu

C_IN = 12
C16 = 16
C_OUT = 32
K = 5
PAD = 2
LG = 4
HIDDEN = 32
OUT = 5
OUT_PAD = 128
TB = 512
GX_CHUNKS = 4


def _ecg_kernel(x_ref, w4_ref, wiht_ref, biht_ref, whht_ref, bhhnt_ref,
                fcw_ref, fcb_ref, out_ref, gx_scr):
    L, _, tb = x_ref.shape
    T = L // 2
    H = HIDDEN
    G6 = 6 * H
    w4 = w4_ref[...]

    zplane = jnp.zeros((C16, tb), jnp.float32)
    feats = []
    for g in range(L // LG):
        planes = [zplane if (l < 0 or l >= L) else x_ref[l]
                  for l in range(LG * g - PAD, LG * g + (K - PAD - 1) + LG)]
        bg = jnp.concatenate([planes[gi + k] for gi in range(LG)
                              for k in range(K)], axis=0)
        yg = jnp.maximum(jnp.dot(w4, bg,
                                 preferred_element_type=jnp.float32), 0.0)
        for p in range(LG // 2):
            feats.append(jnp.maximum(yg[(2 * p) * C_OUT:(2 * p + 1) * C_OUT],
                                     yg[(2 * p + 1) * C_OUT:(2 * p + 2) * C_OUT]))

    wiht = wiht_ref[...]
    tpc = T // GX_CHUNKS
    bih_rep = jnp.concatenate([biht_ref[...]] * (tpc * tb // 128), axis=1)
    for c in range(GX_CHUNKS):
        fc = jnp.concatenate(feats[tpc * c:tpc * (c + 1)], axis=1)
        gxc = jnp.dot(wiht, fc, preferred_element_type=jnp.float32) + bih_rep
        for j in range(tpc):
            gx_scr[tpc * c + j] = gxc[:, j * tb:(j + 1) * tb]

    whht = whht_ref[...]
    bhhn = jnp.concatenate([bhhnt_ref[...]] * (tb // 128), axis=1)

    def step(t, h):
        gf = gx_scr[t]
        gb = gx_scr[T - 1 - t]
        gh = jnp.dot(whht, h, preferred_element_type=jnp.float32)
        rz = jax.nn.sigmoid(
            jnp.concatenate([gf[0:H], gb[H:2 * H],
                             gf[2 * H:3 * H], gb[3 * H:4 * H]], axis=0)
            + gh[:4 * H])
        r = rz[:2 * H]
        z = rz[2 * H:]
        n = jnp.tanh(jnp.concatenate([gf[4 * H:5 * H], gb[5 * H:]], axis=0)
                     + r * (gh[4 * H:] + bhhn))
        return (1.0 - z) * n + z * h

    h0 = jnp.zeros((2 * H, tb), jnp.float32)
    h = jax.lax.fori_loop(0, T, step, h0, unroll=4)

    ht = jnp.transpose(h)
    out_ref[...] = jnp.dot(ht, fcw_ref[...],
                           preferred_element_type=jnp.float32) + fcb_ref[...]


def kernel(x, conv_w, conv_b, wih, bih, whh, bhh_n, fc_w, fc_b):
    B, C, L = x.shape
    assert C == C_IN and L % (2 * LG) == 0
    tb = TB if B % TB == 0 else B
    G = B // tb
    T = L // 2

    xe = jnp.concatenate(
        [x, jnp.ones((B, 1, L), jnp.float32),
         jnp.zeros((B, C16 - C_IN - 1, L), jnp.float32)], axis=1)
    xt = jnp.transpose(xe, (2, 1, 0))

    wk = jnp.transpose(conv_w.reshape(K, C_IN, C_OUT), (0, 2, 1))
    bias_col = jnp.zeros((K, C_OUT, 1), jnp.float32).at[PAD, :, 0].set(
        conv_b[0])
    wk16 = jnp.concatenate(
        [wk, bias_col, jnp.zeros((K, C_OUT, C16 - C_IN - 1), jnp.float32)],
        axis=2)
    wkc = jnp.transpose(wk16, (1, 0, 2)).reshape(C_OUT, K * C16)
    w4 = jnp.zeros((LG * C_OUT, LG * K * C16), jnp.float32)
    for gi in range(LG):
        w4 = w4.at[gi * C_OUT:(gi + 1) * C_OUT,
                   gi * K * C16:(gi + 1) * K * C16].set(wkc)

    wiht = jnp.transpose(wih)
    biht = jnp.broadcast_to(jnp.transpose(bih), (6 * HIDDEN, 128))
    whht = jnp.transpose(whh)
    bhhnt = jnp.broadcast_to(jnp.transpose(bhh_n), (2 * HIDDEN, 128))

    args = (xt, w4, wiht, biht, whht, bhhnt, fc_w, fc_b)

    def full_spec(a):
        return pl.BlockSpec(a.shape, lambda g, nd=a.ndim: (0,) * nd)

    in_specs = ([pl.BlockSpec((L, C16, tb), lambda g: (0, 0, g))]
                + [full_spec(a) for a in args[1:]])

    out = pl.pallas_call(
        _ecg_kernel,
        out_shape=jax.ShapeDtypeStruct((B, OUT_PAD), jnp.float32),
        grid_spec=pltpu.PrefetchScalarGridSpec(
            num_scalar_prefetch=0,
            grid=(G,),
            in_specs=in_specs,
            out_specs=pl.BlockSpec((tb, OUT_PAD), lambda g: (g, 0)),
            scratch_shapes=[pltpu.VMEM((T, 6 * HIDDEN, tb), jnp.float32)],
        ),
        compiler_params=pltpu.CompilerParams(
            dimension_semantics=("parallel",),
            vmem_limit_bytes=100 * 1024 * 1024,
        ),
    )(*args)
    return out[:, :OUT]


# speedup vs baseline: 5.8981x; 1.0775x over previous
import jax
import jax.numpy as jnp
from jax.experimental import pallas as pl
from jax.experimental.pallas import tpu as pltpu

C_IN = 12
C16 = 16
C_OUT = 32
K = 5
PAD = 2
LG = 4
HIDDEN = 32
OUT = 5
OUT_PAD = 128
TB = 512
GX_CHUNKS = 4


def _ecg_kernel(x_ref, w4_ref, wiht_ref, whhft_ref, whhbt_ref, bhhnt_ref,
                fcw_ref, fcb_ref, out_ref, gx_scr):
    L, _, tb = x_ref.shape
    T = L // 2
    H = HIDDEN
    G6 = 6 * H
    w4 = w4_ref[...]

    zplane = jnp.zeros((C16, tb), jnp.float32)
    feats = []
    for g in range(L // LG):
        planes = [zplane if (l < 0 or l >= L) else x_ref[l]
                  for l in range(LG * g - PAD, LG * g + (K - PAD - 1) + LG)]
        bg = jnp.concatenate([planes[gi + k] for gi in range(LG)
                              for k in range(K)], axis=0)
        yg = jnp.maximum(jnp.dot(w4, bg,
                                 preferred_element_type=jnp.float32), 0.0)
        for p in range(LG // 2):
            feats.append(jnp.maximum(yg[(2 * p) * C_OUT:(2 * p + 1) * C_OUT],
                                     yg[(2 * p + 1) * C_OUT:(2 * p + 2) * C_OUT]))

    wiht = wiht_ref[...]
    tpc = T // GX_CHUNKS
    ones8 = jnp.ones((8, tb), jnp.float32)
    for c in range(GX_CHUNKS):
        fc = jnp.concatenate(
            [jnp.concatenate([f, ones8], axis=0)
             for f in feats[tpc * c:tpc * (c + 1)]], axis=1)
        gxc = jnp.dot(wiht, fc, preferred_element_type=jnp.float32)
        for j in range(tpc):
            gx_scr[tpc * c + j] = gxc[:, j * tb:(j + 1) * tb]

    whhf = whhft_ref[...]
    whhb = whhbt_ref[...]
    bhhn2 = jnp.concatenate([bhhnt_ref[...]] * (tb // 128), axis=1)
    bhf = bhhn2[:H]
    bhb = bhhn2[H:]

    def step(t, hs):
        hf, hb = hs
        gf = gx_scr[t]
        gb = gx_scr[T - 1 - t]
        ghf = jnp.dot(whhf, hf, preferred_element_type=jnp.float32)
        ghb = jnp.dot(whhb, hb, preferred_element_type=jnp.float32)
        rzf = jax.nn.sigmoid(
            jnp.concatenate([gf[0:H], gf[2 * H:3 * H]], axis=0) + ghf[:2 * H])
        rzb = jax.nn.sigmoid(
            jnp.concatenate([gb[H:2 * H], gb[3 * H:4 * H]], axis=0)
            + ghb[:2 * H])
        nf = jnp.tanh(gf[4 * H:5 * H] + rzf[:H] * (ghf[2 * H:] + bhf))
        nb = jnp.tanh(gb[5 * H:] + rzb[:H] * (ghb[2 * H:] + bhb))
        hf = (1.0 - rzf[H:]) * nf + rzf[H:] * hf
        hb = (1.0 - rzb[H:]) * nb + rzb[H:] * hb
        return (hf, hb)

    h0 = jnp.zeros((H, tb), jnp.float32)
    hf, hb = jax.lax.fori_loop(0, T, step, (h0, h0), unroll=4)

    ht = jnp.transpose(jnp.concatenate([hf, hb], axis=0))
    out_ref[...] = jnp.dot(ht, fcw_ref[...],
                           preferred_element_type=jnp.float32) + fcb_ref[...]


def kernel(x, conv_w, conv_b, wih, bih, whh, bhh_n, fc_w, fc_b):
    B, C, L = x.shape
    assert C == C_IN and L % (2 * LG) == 0
    tb = TB if B % TB == 0 else B
    G = B // tb
    T = L // 2

    xe = jnp.concatenate(
        [x, jnp.ones((B, 1, L), jnp.float32),
         jnp.zeros((B, C16 - C_IN - 1, L), jnp.float32)], axis=1)
    xt = jnp.transpose(xe, (2, 1, 0))

    wk = jnp.transpose(conv_w.reshape(K, C_IN, C_OUT), (0, 2, 1))
    bias_col = jnp.zeros((K, C_OUT, 1), jnp.float32).at[PAD, :, 0].set(
        conv_b[0])
    wk16 = jnp.concatenate(
        [wk, bias_col, jnp.zeros((K, C_OUT, C16 - C_IN - 1), jnp.float32)],
        axis=2)
    wkc = jnp.transpose(wk16, (1, 0, 2)).reshape(C_OUT, K * C16)
    w4 = jnp.zeros((LG * C_OUT, LG * K * C16), jnp.float32)
    for gi in range(LG):
        w4 = w4.at[gi * C_OUT:(gi + 1) * C_OUT,
                   gi * K * C16:(gi + 1) * K * C16].set(wkc)

    H = HIDDEN
    wiht = jnp.concatenate(
        [jnp.transpose(wih), jnp.transpose(bih),
         jnp.zeros((6 * H, 7), jnp.float32)], axis=1)
    whhft = jnp.concatenate(
        [jnp.transpose(whh[0:H, 0 * H:1 * H]),
         jnp.transpose(whh[0:H, 2 * H:3 * H]),
         jnp.transpose(whh[0:H, 4 * H:5 * H])], axis=0)
    whhbt = jnp.concatenate(
        [jnp.transpose(whh[H:2 * H, 1 * H:2 * H]),
         jnp.transpose(whh[H:2 * H, 3 * H:4 * H]),
         jnp.transpose(whh[H:2 * H, 5 * H:6 * H])], axis=0)
    bhhnt = jnp.broadcast_to(jnp.transpose(bhh_n), (2 * HIDDEN, 128))

    args = (xt, w4, wiht, whhft, whhbt, bhhnt, fc_w, fc_b)

    def full_spec(a):
        return pl.BlockSpec(a.shape, lambda g, nd=a.ndim: (0,) * nd)

    in_specs = ([pl.BlockSpec((L, C16, tb), lambda g: (0, 0, g))]
                + [full_spec(a) for a in args[1:]])

    out = pl.pallas_call(
        _ecg_kernel,
        out_shape=jax.ShapeDtypeStruct((B, OUT_PAD), jnp.float32),
        grid_spec=pltpu.PrefetchScalarGridSpec(
            num_scalar_prefetch=0,
            grid=(G,),
            in_specs=in_specs,
            out_specs=pl.BlockSpec((tb, OUT_PAD), lambda g: (g, 0)),
            scratch_shapes=[pltpu.VMEM((T, 6 * HIDDEN, tb), jnp.float32)],
        ),
        compiler_params=pltpu.CompilerParams(
            dimension_semantics=("parallel",),
            vmem_limit_bytes=100 * 1024 * 1024,
        ),
    )(*args)
    return out[:, :OUT]


# speedup vs baseline: 6.0410x; 1.0242x over previous
import jax
import jax.numpy as jnp
from jax.experimental import pallas as pl
from jax.experimental.pallas import tpu as pltpu

C_IN = 12
C16 = 16
C_OUT = 32
K = 5
PAD = 2
LG = 4
HIDDEN = 32
OUT = 5
OUT_PAD = 128
TB = 512
GX_CHUNKS = 4


def _ecg_kernel(x_ref, w4_ref, wiht_ref, whhft_ref, whhbt_ref, bhhnt_ref,
                fcw_ref, fcb_ref, out_ref, gx_scr):
    L, _, tb = x_ref.shape
    T = L // 2
    H = HIDDEN
    G6 = 6 * H
    w4 = w4_ref[...]

    zplane = jnp.zeros((C16, tb), jnp.bfloat16)
    feats = []
    for g in range(L // LG):
        planes = [zplane if (l < 0 or l >= L) else x_ref[l]
                  for l in range(LG * g - PAD, LG * g + (K - PAD - 1) + LG)]
        bg = jnp.concatenate([planes[gi + k] for gi in range(LG)
                              for k in range(K)], axis=0)
        yg = jnp.maximum(jnp.dot(w4, bg,
                                 preferred_element_type=jnp.float32), 0.0)
        for p in range(LG // 2):
            feats.append(jnp.maximum(yg[(2 * p) * C_OUT:(2 * p + 1) * C_OUT],
                                     yg[(2 * p + 1) * C_OUT:(2 * p + 2) * C_OUT]))

    wiht = wiht_ref[...]
    tpc = T // GX_CHUNKS
    ones8 = jnp.ones((8, tb), jnp.bfloat16)
    for c in range(GX_CHUNKS):
        fc = jnp.concatenate(
            [jnp.concatenate([f.astype(jnp.bfloat16), ones8], axis=0)
             for f in feats[tpc * c:tpc * (c + 1)]], axis=1)
        gxc = jnp.dot(wiht, fc, preferred_element_type=jnp.float32)
        for j in range(tpc):
            gx_scr[tpc * c + j] = gxc[:, j * tb:(j + 1) * tb].astype(
                jnp.bfloat16)

    whhf = whhft_ref[...]
    whhb = whhbt_ref[...]
    bhhn2 = jnp.concatenate([bhhnt_ref[...]] * (tb // 128), axis=1)
    bhf = bhhn2[:H]
    bhb = bhhn2[H:]

    def step(t, hs):
        hf, hb = hs
        gf = gx_scr[t]
        gb = gx_scr[T - 1 - t]
        ghf = jnp.dot(whhf, hf, preferred_element_type=jnp.float32)
        ghb = jnp.dot(whhb, hb, preferred_element_type=jnp.float32)
        rzf = jax.nn.sigmoid(
            jnp.concatenate([gf[0:H], gf[2 * H:3 * H]], axis=0) + ghf[:2 * H])
        rzb = jax.nn.sigmoid(
            jnp.concatenate([gb[H:2 * H], gb[3 * H:4 * H]], axis=0)
            + ghb[:2 * H])
        nf = jnp.tanh(gf[4 * H:5 * H] + rzf[:H] * (ghf[2 * H:] + bhf))
        nb = jnp.tanh(gb[5 * H:] + rzb[:H] * (ghb[2 * H:] + bhb))
        hf = (1.0 - rzf[H:]) * nf + rzf[H:] * hf
        hb = (1.0 - rzb[H:]) * nb + rzb[H:] * hb
        return (hf, hb)

    h0 = jnp.zeros((H, tb), jnp.float32)
    hf, hb = jax.lax.fori_loop(0, T, step, (h0, h0), unroll=4)

    ht = jnp.transpose(jnp.concatenate([hf, hb], axis=0))
    out_ref[...] = jnp.dot(ht, fcw_ref[...],
                           preferred_element_type=jnp.float32) + fcb_ref[...]


def kernel(x, conv_w, conv_b, wih, bih, whh, bhh_n, fc_w, fc_b):
    B, C, L = x.shape
    assert C == C_IN and L % (2 * LG) == 0
    tb = TB if B % TB == 0 else B
    G = B // tb
    T = L // 2

    xe = jnp.concatenate(
        [x, jnp.ones((B, 1, L), jnp.float32),
         jnp.zeros((B, C16 - C_IN - 1, L), jnp.float32)], axis=1)
    xt = jnp.transpose(xe, (2, 1, 0)).astype(jnp.bfloat16)

    wk = jnp.transpose(conv_w.reshape(K, C_IN, C_OUT), (0, 2, 1))
    bias_col = jnp.zeros((K, C_OUT, 1), jnp.float32).at[PAD, :, 0].set(
        conv_b[0])
    wk16 = jnp.concatenate(
        [wk, bias_col, jnp.zeros((K, C_OUT, C16 - C_IN - 1), jnp.float32)],
        axis=2)
    wkc = jnp.transpose(wk16, (1, 0, 2)).reshape(C_OUT, K * C16)
    w4 = jnp.zeros((LG * C_OUT, LG * K * C16), jnp.float32)
    for gi in range(LG):
        w4 = w4.at[gi * C_OUT:(gi + 1) * C_OUT,
                   gi * K * C16:(gi + 1) * K * C16].set(wkc)
    w4 = w4.astype(jnp.bfloat16)

    H = HIDDEN
    wiht = jnp.concatenate(
        [jnp.transpose(wih), jnp.transpose(bih),
         jnp.zeros((6 * H, 7), jnp.float32)], axis=1).astype(
             jnp.bfloat16)
    whhft = jnp.concatenate(
        [jnp.transpose(whh[0:H, 0 * H:1 * H]),
         jnp.transpose(whh[0:H, 2 * H:3 * H]),
         jnp.transpose(whh[0:H, 4 * H:5 * H])], axis=0)
    whhbt = jnp.concatenate(
        [jnp.transpose(whh[H:2 * H, 1 * H:2 * H]),
         jnp.transpose(whh[H:2 * H, 3 * H:4 * H]),
         jnp.transpose(whh[H:2 * H, 5 * H:6 * H])], axis=0)
    bhhnt = jnp.broadcast_to(jnp.transpose(bhh_n), (2 * HIDDEN, 128))

    args = (xt, w4, wiht, whhft, whhbt, bhhnt, fc_w, fc_b)

    def full_spec(a):
        return pl.BlockSpec(a.shape, lambda g, nd=a.ndim: (0,) * nd)

    in_specs = ([pl.BlockSpec((L, C16, tb), lambda g: (0, 0, g))]
                + [full_spec(a) for a in args[1:]])

    out = pl.pallas_call(
        _ecg_kernel,
        out_shape=jax.ShapeDtypeStruct((B, OUT_PAD), jnp.float32),
        grid_spec=pltpu.PrefetchScalarGridSpec(
            num_scalar_prefetch=0,
            grid=(G,),
            in_specs=in_specs,
            out_specs=pl.BlockSpec((tb, OUT_PAD), lambda g: (g, 0)),
            scratch_shapes=[pltpu.VMEM((T, 6 * HIDDEN, tb), jnp.bfloat16)],
        ),
        compiler_params=pltpu.CompilerParams(
            dimension_semantics=("parallel",),
            vmem_limit_bytes=100 * 1024 * 1024,
        ),
    )(*args)
    return out[:, :OUT]


# speedup vs baseline: 7.3357x; 1.2143x over previous
import jax
import jax.numpy as jnp
from jax.experimental import pallas as pl
from jax.experimental.pallas import tpu as pltpu

C_IN = 12
C16 = 16
C_OUT = 32
K = 5
PAD = 2
LG = 4
HIDDEN = 32
OUT = 5
OUT_PAD = 128
TB = 1024
GX_CHUNKS = 8


def _ecg_kernel(x_ref, w4_ref, wiht_ref, whhft_ref, whhbt_ref, bhhnt_ref,
                fcw_ref, fcb_ref, out_ref, gx_scr):
    L, _, tb = x_ref.shape
    T = L // 2
    H = HIDDEN
    G6 = 6 * H
    w4 = w4_ref[...]

    zplane = jnp.zeros((C16, tb), jnp.bfloat16)
    feats = []
    for g in range(L // LG):
        planes = [zplane if (l < 0 or l >= L) else x_ref[l]
                  for l in range(LG * g - PAD, LG * g + (K - PAD - 1) + LG)]
        bg = jnp.concatenate([planes[gi + k] for gi in range(LG)
                              for k in range(K)], axis=0)
        yg = jnp.maximum(jnp.dot(w4, bg,
                                 preferred_element_type=jnp.float32), 0.0)
        for p in range(LG // 2):
            feats.append(jnp.maximum(yg[(2 * p) * C_OUT:(2 * p + 1) * C_OUT],
                                     yg[(2 * p + 1) * C_OUT:(2 * p + 2) * C_OUT]))

    wiht = wiht_ref[...]
    tpc = T // GX_CHUNKS
    ones8 = jnp.ones((8, tb), jnp.bfloat16)
    for c in range(GX_CHUNKS):
        fc = jnp.concatenate(
            [jnp.concatenate([f.astype(jnp.bfloat16), ones8], axis=0)
             for f in feats[tpc * c:tpc * (c + 1)]], axis=1)
        gxc = jnp.dot(wiht, fc, preferred_element_type=jnp.float32)
        for j in range(tpc):
            gx_scr[tpc * c + j] = gxc[:, j * tb:(j + 1) * tb].astype(
                jnp.bfloat16)

    whhf = whhft_ref[...]
    whhb = whhbt_ref[...]
    bhhn2 = jnp.concatenate([bhhnt_ref[...]] * (tb // 128), axis=1)
    bhf = bhhn2[:H]
    bhb = bhhn2[H:]

    def step(t, hs):
        hf, hb = hs
        gf = gx_scr[t]
        gb = gx_scr[T - 1 - t]
        ghf = jnp.dot(whhf, hf, preferred_element_type=jnp.float32)
        ghb = jnp.dot(whhb, hb, preferred_element_type=jnp.float32)
        rzf = jax.nn.sigmoid(
            jnp.concatenate([gf[0:H], gf[2 * H:3 * H]], axis=0) + ghf[:2 * H])
        rzb = jax.nn.sigmoid(
            jnp.concatenate([gb[H:2 * H], gb[3 * H:4 * H]], axis=0)
            + ghb[:2 * H])
        nf = jnp.tanh(gf[4 * H:5 * H] + rzf[:H] * (ghf[2 * H:] + bhf))
        nb = jnp.tanh(gb[5 * H:] + rzb[:H] * (ghb[2 * H:] + bhb))
        hf = (1.0 - rzf[H:]) * nf + rzf[H:] * hf
        hb = (1.0 - rzb[H:]) * nb + rzb[H:] * hb
        return (hf, hb)

    h0 = jnp.zeros((H, tb), jnp.float32)
    hf, hb = jax.lax.fori_loop(0, T, step, (h0, h0), unroll=4)

    ht = jnp.transpose(jnp.concatenate([hf, hb], axis=0))
    out_ref[...] = jnp.dot(ht, fcw_ref[...],
                           preferred_element_type=jnp.float32) + fcb_ref[...]


def kernel(x, conv_w, conv_b, wih, bih, whh, bhh_n, fc_w, fc_b):
    B, C, L = x.shape
    assert C == C_IN and L % (2 * LG) == 0
    tb = TB if B % TB == 0 else B
    G = B // tb
    T = L // 2

    xe = jnp.concatenate(
        [x, jnp.ones((B, 1, L), jnp.float32),
         jnp.zeros((B, C16 - C_IN - 1, L), jnp.float32)], axis=1)
    xt = jnp.transpose(xe, (2, 1, 0)).astype(jnp.bfloat16)

    wk = jnp.transpose(conv_w.reshape(K, C_IN, C_OUT), (0, 2, 1))
    bias_col = jnp.zeros((K, C_OUT, 1), jnp.float32).at[PAD, :, 0].set(
        conv_b[0])
    wk16 = jnp.concatenate(
        [wk, bias_col, jnp.zeros((K, C_OUT, C16 - C_IN - 1), jnp.float32)],
        axis=2)
    wkc = jnp.transpose(wk16, (1, 0, 2)).reshape(C_OUT, K * C16)
    w4 = jnp.zeros((LG * C_OUT, LG * K * C16), jnp.float32)
    for gi in range(LG):
        w4 = w4.at[gi * C_OUT:(gi + 1) * C_OUT,
                   gi * K * C16:(gi + 1) * K * C16].set(wkc)
    w4 = w4.astype(jnp.bfloat16)

    H = HIDDEN
    wiht = jnp.concatenate(
        [jnp.transpose(wih), jnp.transpose(bih),
         jnp.zeros((6 * H, 7), jnp.float32)], axis=1).astype(
             jnp.bfloat16)
    whhft = jnp.concatenate(
        [jnp.transpose(whh[0:H, 0 * H:1 * H]),
         jnp.transpose(whh[0:H, 2 * H:3 * H]),
         jnp.transpose(whh[0:H, 4 * H:5 * H])], axis=0)
    whhbt = jnp.concatenate(
        [jnp.transpose(whh[H:2 * H, 1 * H:2 * H]),
         jnp.transpose(whh[H:2 * H, 3 * H:4 * H]),
         jnp.transpose(whh[H:2 * H, 5 * H:6 * H])], axis=0)
    bhhnt = jnp.broadcast_to(jnp.transpose(bhh_n), (2 * HIDDEN, 128))

    args = (xt, w4, wiht, whhft, whhbt, bhhnt, fc_w, fc_b)

    def full_spec(a):
        return pl.BlockSpec(a.shape, lambda g, nd=a.ndim: (0,) * nd)

    in_specs = ([pl.BlockSpec((L, C16, tb), lambda g: (0, 0, g))]
                + [full_spec(a) for a in args[1:]])

    out = pl.pallas_call(
        _ecg_kernel,
        out_shape=jax.ShapeDtypeStruct((B, OUT_PAD), jnp.float32),
        grid_spec=pltpu.PrefetchScalarGridSpec(
            num_scalar_prefetch=0,
            grid=(G,),
            in_specs=in_specs,
            out_specs=pl.BlockSpec((tb, OUT_PAD), lambda g: (g, 0)),
            scratch_shapes=[pltpu.VMEM((T, 6 * HIDDEN, tb), jnp.bfloat16)],
        ),
        compiler_params=pltpu.CompilerParams(
            dimension_semantics=("parallel",),
            vmem_limit_bytes=100 * 1024 * 1024,
        ),
    )(*args)
    return out[:, :OUT]


# speedup vs baseline: 8.3175x; 1.1338x over previous
import jax
import jax.numpy as jnp
from jax.experimental import pallas as pl
from jax.experimental.pallas import tpu as pltpu

C_IN = 12
C16 = 16
C_OUT = 32
K = 5
PAD = 2
LG = 4
HIDDEN = 32
OUT = 5
OUT_PAD = 128
TB = 1024
GX_CHUNKS = 8


def _ecg_kernel(x_ref, w4_ref, wihft_ref, wihbt_ref, whhft_ref, whhbt_ref,
                bhhnt_ref, fcw_ref, fcb_ref, out_ref, f_scr):
    L, _, tb = x_ref.shape
    T = L // 2
    H = HIDDEN
    G6 = 6 * H
    w4 = w4_ref[...]

    zplane = jnp.zeros((C16, tb), jnp.bfloat16)
    ones8 = jnp.ones((8, tb), jnp.bfloat16)
    for g in range(L // LG):
        planes = [zplane if (l < 0 or l >= L) else x_ref[l]
                  for l in range(LG * g - PAD, LG * g + (K - PAD - 1) + LG)]
        bg = jnp.concatenate([planes[gi + k] for gi in range(LG)
                              for k in range(K)], axis=0)
        yg = jnp.maximum(jnp.dot(w4, bg,
                                 preferred_element_type=jnp.float32), 0.0)
        for p in range(LG // 2):
            f = jnp.maximum(yg[(2 * p) * C_OUT:(2 * p + 1) * C_OUT],
                            yg[(2 * p + 1) * C_OUT:(2 * p + 2) * C_OUT])
            f_scr[2 * g + p] = jnp.concatenate(
                [f.astype(jnp.bfloat16), ones8], axis=0)

    wihf = wihft_ref[...]
    wihb = wihbt_ref[...]
    whhf = whhft_ref[...]
    whhb = whhbt_ref[...]
    bhhn2 = jnp.concatenate([bhhnt_ref[...]] * (tb // 128), axis=1)
    bhf = bhhn2[:H]
    bhb = bhhn2[H:]

    def step(t, hs):
        hf, hb = hs
        gxf = jnp.dot(wihf, f_scr[t],
                      preferred_element_type=jnp.float32)
        gxb = jnp.dot(wihb, f_scr[T - 1 - t],
                      preferred_element_type=jnp.float32)
        ghf = jnp.dot(whhf, hf, preferred_element_type=jnp.float32)
        ghb = jnp.dot(whhb, hb, preferred_element_type=jnp.float32)
        rzf = jax.nn.sigmoid(gxf[:2 * H] + ghf[:2 * H])
        rzb = jax.nn.sigmoid(gxb[:2 * H] + ghb[:2 * H])
        nf = jnp.tanh(gxf[2 * H:] + rzf[:H] * (ghf[2 * H:] + bhf))
        nb = jnp.tanh(gxb[2 * H:] + rzb[:H] * (ghb[2 * H:] + bhb))
        hf = (1.0 - rzf[H:]) * nf + rzf[H:] * hf
        hb = (1.0 - rzb[H:]) * nb + rzb[H:] * hb
        return (hf, hb)

    h0 = jnp.zeros((H, tb), jnp.float32)
    hf, hb = jax.lax.fori_loop(0, T, step, (h0, h0), unroll=4)

    ht = jnp.transpose(jnp.concatenate([hf, hb], axis=0))
    out_ref[...] = jnp.dot(ht, fcw_ref[...],
                           preferred_element_type=jnp.float32) + fcb_ref[...]


def kernel(x, conv_w, conv_b, wih, bih, whh, bhh_n, fc_w, fc_b):
    B, C, L = x.shape
    assert C == C_IN and L % (2 * LG) == 0
    tb = TB if B % TB == 0 else B
    G = B // tb
    T = L // 2

    xe = jnp.concatenate(
        [x, jnp.ones((B, 1, L), jnp.float32),
         jnp.zeros((B, C16 - C_IN - 1, L), jnp.float32)], axis=1)
    xt = jnp.transpose(xe, (2, 1, 0)).astype(jnp.bfloat16)

    wk = jnp.transpose(conv_w.reshape(K, C_IN, C_OUT), (0, 2, 1))
    bias_col = jnp.zeros((K, C_OUT, 1), jnp.float32).at[PAD, :, 0].set(
        conv_b[0])
    wk16 = jnp.concatenate(
        [wk, bias_col, jnp.zeros((K, C_OUT, C16 - C_IN - 1), jnp.float32)],
        axis=2)
    wkc = jnp.transpose(wk16, (1, 0, 2)).reshape(C_OUT, K * C16)
    w4 = jnp.zeros((LG * C_OUT, LG * K * C16), jnp.float32)
    for gi in range(LG):
        w4 = w4.at[gi * C_OUT:(gi + 1) * C_OUT,
                   gi * K * C16:(gi + 1) * K * C16].set(wkc)
    w4 = w4.astype(jnp.bfloat16)

    H = HIDDEN
    wt = jnp.transpose(wih)
    bt = jnp.transpose(bih)
    z7 = jnp.zeros((3 * H, 7), jnp.float32)

    def _wih_dir(off):
        rows = [slice(off * H, (off + 1) * H),
                slice((2 + off) * H, (3 + off) * H),
                slice((4 + off) * H, (5 + off) * H)]
        w = jnp.concatenate([wt[s] for s in rows], axis=0)
        b = jnp.concatenate([bt[s] for s in rows], axis=0)
        return jnp.concatenate([w, b, z7], axis=1).astype(jnp.bfloat16)

    wihft = _wih_dir(0)
    wihbt = _wih_dir(1)
    whhft = jnp.concatenate(
        [jnp.transpose(whh[0:H, 0 * H:1 * H]),
         jnp.transpose(whh[0:H, 2 * H:3 * H]),
         jnp.transpose(whh[0:H, 4 * H:5 * H])], axis=0)
    whhbt = jnp.concatenate(
        [jnp.transpose(whh[H:2 * H, 1 * H:2 * H]),
         jnp.transpose(whh[H:2 * H, 3 * H:4 * H]),
         jnp.transpose(whh[H:2 * H, 5 * H:6 * H])], axis=0)
    bhhnt = jnp.broadcast_to(jnp.transpose(bhh_n), (2 * HIDDEN, 128))

    args = (xt, w4, wihft, wihbt, whhft, whhbt, bhhnt, fc_w, fc_b)

    def full_spec(a):
        return pl.BlockSpec(a.shape, lambda g, nd=a.ndim: (0,) * nd)

    in_specs = ([pl.BlockSpec((L, C16, tb), lambda g: (0, 0, g))]
                + [full_spec(a) for a in args[1:]])

    out = pl.pallas_call(
        _ecg_kernel,
        out_shape=jax.ShapeDtypeStruct((B, OUT_PAD), jnp.float32),
        grid_spec=pltpu.PrefetchScalarGridSpec(
            num_scalar_prefetch=0,
            grid=(G,),
            in_specs=in_specs,
            out_specs=pl.BlockSpec((tb, OUT_PAD), lambda g: (g, 0)),
            scratch_shapes=[pltpu.VMEM((T, 40, tb), jnp.bfloat16)],
        ),
        compiler_params=pltpu.CompilerParams(
            dimension_semantics=("parallel",),
            vmem_limit_bytes=100 * 1024 * 1024,
        ),
    )(*args)
    return out[:, :OUT]
